# baseline (speedup 1.0000x reference)
import functools

import jax
import jax.numpy as jnp
from jax import lax
from jax.experimental import pallas as pl
from jax.experimental.pallas import tpu as pltpu
from jax.experimental.pallas import tpu_sc as plsc

N = 10000
NP = 10240
E = 320000
EP = 327680
NB32 = 80
CH = 16
B = 128
RT = NP // 16
F32 = jnp.float32

@functools.lru_cache(maxsize=1)
def _mesh():
    return plsc.VectorSubcoreMesh(core_axis_name="c", subcore_axis_name="s",
                                  num_cores=2, num_subcores=16)


def _zero_vmem(ref, nrows, ncols):
    z = jnp.zeros((16,), F32)
    def body(r, _):
        for j in range(ncols // 16):
            ref[r, pl.ds(j * 16, 16)] = z
        return 0
    lax.fori_loop(0, nrows, body, 0, unroll=False)


def _deg_body(dsts_hbm, out_hbm, dst_v, vals_v, dacc, sem):
    cid = lax.axis_index("c")
    sid = lax.axis_index("s")
    _zero_vmem(vals_v, B, B)
    for k in range(RT // B):
        pltpu.sync_copy(vals_v, dacc.at[pl.ds(sid * RT + k * B, B)])
    one = jnp.ones((16,), F32)
    def fill(r, _):
        for j in range(B // 16):
            vals_v[r, pl.ds(j * 16, 16)] = one
        return 0
    lax.fori_loop(0, B, fill, 0, unroll=False)
    pltpu.sync_copy(dsts_hbm.at[sid * 2 + cid], dst_v)
    plsc.subcore_barrier()
    def body(b, _):
        pltpu.sync_copy(vals_v, dacc.at[dst_v.at[b]], add=True)
        return 0
    lax.fori_loop(0, NB32, body, 0, unroll=False)
    plsc.subcore_barrier()
    pltpu.sync_copy(dacc.at[pl.ds(sid * RT, RT)],
                    out_hbm.at[pl.ds(cid * NP + sid * RT, RT)])


def _sc_deg(dsts):
    f = pl.kernel(
        _deg_body,
        out_type=jax.ShapeDtypeStruct((2 * NP, B), F32),
        mesh=_mesh(),
        scratch_types=[
            pltpu.VMEM((NB32, B), jnp.int32),
            pltpu.VMEM((B, B), F32),
            pltpu.VMEM_SHARED((NP, B), F32),
            pltpu.SemaphoreType.DMA,
        ],
    )
    return f(dsts)


def _scat_body(g_hbm, srcs_hbm, dsts_hbm, out_hbm, src_v, dst_v, rows_v, acc,
               sem_g0, sem_g1, sem_s0, sem_s1):
    cid = lax.axis_index("c")
    sid = lax.axis_index("s")
    z = jnp.zeros((16,), F32)
    def zr(r, _):
        for j in range(B // 16):
            rows_v[0, r, pl.ds(j * 16, 16)] = z
        return 0
    lax.fori_loop(0, B, zr, 0, unroll=False)
    for k in range(RT // B):
        pltpu.sync_copy(rows_v.at[0], acc.at[pl.ds(sid * RT + k * B, B)])
    plsc.subcore_barrier()
    buf = (rows_v.at[0], rows_v.at[1])
    sg = (sem_g0, sem_g1)
    ss = (sem_s0, sem_s1)
    for part in range(2):
        def chunk(ch, _, part=part):
            pltpu.sync_copy(
                srcs_hbm.at[cid, sid * 2 + part, pl.ds(ch * CH, CH)], src_v)
            pltpu.sync_copy(
                dsts_hbm.at[sid * 2 + part, pl.ds(ch * CH, CH)], dst_v)
            pltpu.async_copy(g_hbm.at[src_v.at[0]], buf[0], sg[0])
            def body(k2, _):
                k = 2 * k2
                @pl.when(k2 >= 1)
                def _():
                    pltpu.make_async_copy(buf[1], acc.at[dst_v.at[k - 1]],
                                          ss[1]).wait()
                pltpu.async_copy(g_hbm.at[src_v.at[k + 1]], buf[1], sg[1])
                pltpu.make_async_copy(g_hbm.at[src_v.at[k]], buf[0],
                                      sg[0]).wait()
                pltpu.async_copy(buf[0], acc.at[dst_v.at[k]], ss[0], add=True)
                @pl.when(k2 < CH // 2 - 1)
                def _():
                    pltpu.make_async_copy(buf[0], acc.at[dst_v.at[k]],
                                          ss[0]).wait()
                    pltpu.async_copy(g_hbm.at[src_v.at[k + 2]], buf[0], sg[0])
                pltpu.make_async_copy(g_hbm.at[src_v.at[k + 1]], buf[1],
                                      sg[1]).wait()
                pltpu.async_copy(buf[1], acc.at[dst_v.at[k + 1]], ss[1],
                                 add=True)
                return 0
            lax.fori_loop(0, CH // 2, body, 0, unroll=False)
            pltpu.make_async_copy(buf[0], acc.at[dst_v.at[CH - 2]],
                                  ss[0]).wait()
            pltpu.make_async_copy(buf[1], acc.at[dst_v.at[CH - 1]],
                                  ss[1]).wait()
            return 0
        lax.fori_loop(0, NB32 // CH, chunk, 0, unroll=False)
    plsc.subcore_barrier()
    pltpu.sync_copy(acc.at[pl.ds(sid * RT, RT)],
                    out_hbm.at[pl.ds(cid * NP + sid * RT, RT)])


def _sc_scatter(g2, srcs2, dsts):
    f = pl.kernel(
        _scat_body,
        out_type=jax.ShapeDtypeStruct((2 * NP, B), F32),
        mesh=_mesh(),
        scratch_types=[
            pltpu.VMEM((CH, B), jnp.int32),
            pltpu.VMEM((CH, B), jnp.int32),
            pltpu.VMEM((2, B, B), F32),
            pltpu.VMEM_SHARED((NP, B), F32),
            pltpu.SemaphoreType.DMA,
            pltpu.SemaphoreType.DMA,
            pltpu.SemaphoreType.DMA,
            pltpu.SemaphoreType.DMA,
        ],
    )
    return f(g2, srcs2, dsts)


def _dis_block(degp_ref):
    i = pl.program_id(0)
    d = degp_ref[0, :, 0:1] + degp_ref[1, :, 0:1]
    rows = lax.broadcasted_iota(jnp.int32, (1280, 1), 0) + i * 1280
    return jnp.where(rows < N, lax.rsqrt(d + 1.0), 0.0)


_DEGP_SPEC = pl.BlockSpec((2, 1280, 128), lambda i: (0, i, 0))


_DOT = functools.partial(jnp.dot, preferred_element_type=F32,
                         precision=lax.Precision.HIGHEST)


def _pre_body(x_ref, w_ref, b_ref, h_ref):
    h = _DOT(x_ref[...], w_ref[...]) + b_ref[0]
    h_ref[0] = h[:, :128]
    h_ref[1] = h[:, 128:]


def _tc_pre(xp, W_pre, b_pre):
    return pl.pallas_call(
        _pre_body,
        grid=(8,),
        in_specs=[
            pl.BlockSpec((1280, 128), lambda i: (i, 0)),
            pl.BlockSpec((128, 256), lambda i: (0, 0)),
            pl.BlockSpec((1, 256), lambda i: (0, 0)),
        ],
        out_specs=pl.BlockSpec((2, 1280, 128), lambda i: (0, i, 0)),
        out_shape=jax.ShapeDtypeStruct((2, NP, 128), F32),
    )(xp, W_pre, b_pre)


def _mm1_body(h_ref, degp_ref, w_ref, g_ref):
    d = _dis_block(degp_ref)
    g = _DOT(h_ref[0] * d, w_ref[:128, :]) + _DOT(h_ref[1] * d, w_ref[128:, :])
    g_ref[0] = g[:, :128]
    g_ref[1] = g[:, 128:]


def _tc_mm1(h, degp, W):
    return pl.pallas_call(
        _mm1_body,
        grid=(8,),
        in_specs=[
            pl.BlockSpec((2, 1280, 128), lambda i: (0, i, 0)),
            _DEGP_SPEC,
            pl.BlockSpec((256, 256), lambda i: (0, 0)),
        ],
        out_specs=pl.BlockSpec((2, 1280, 128), lambda i: (0, i, 0)),
        out_shape=jax.ShapeDtypeStruct((2, NP, 128), F32),
    )(h, degp, W)


def _fuse_body(agg_ref, g_ref, degp_ref, bc_ref, w_ref, gn_ref):
    d = _dis_block(degp_ref)
    t = []
    for j in range(2):
        u = jnp.maximum(d * (agg_ref[j] + g_ref[j])
                        + bc_ref[0, j * 128:(j + 1) * 128], 0.0)
        t.append(d * u)
    gn = _DOT(t[0], w_ref[:128, :]) + _DOT(t[1], w_ref[128:, :])
    gn_ref[0] = gn[:, :128]
    gn_ref[1] = gn[:, 128:]


def _tc_fuse(agg, g, degp, bc_i, W):
    return pl.pallas_call(
        _fuse_body,
        grid=(8,),
        in_specs=[
            pl.BlockSpec((2, 1280, 128), lambda i: (0, i, 0)),
            pl.BlockSpec((2, 1280, 128), lambda i: (0, i, 0)),
            _DEGP_SPEC,
            pl.BlockSpec((1, 256), lambda i: (0, 0)),
            pl.BlockSpec((256, 256), lambda i: (0, 0)),
        ],
        out_specs=pl.BlockSpec((2, 1280, 128), lambda i: (0, i, 0)),
        out_shape=jax.ShapeDtypeStruct((2, NP, 128), F32),
    )(agg, g, degp, bc_i, W)


def _postepi_body(agg_ref, g_ref, degp_ref, bc_ref, w_ref, b_ref, h_ref,
                  out_ref):
    d = _dis_block(degp_ref)
    u = []
    for j in range(2):
        uj = jnp.maximum(d * (agg_ref[j] + g_ref[j])
                         + bc_ref[0, j * 128:(j + 1) * 128], 0.0)
        h_ref[:, j * 128:(j + 1) * 128] = uj
        u.append(uj)
    logits = (_DOT(u[0], w_ref[:128, :]) + _DOT(u[1], w_ref[128:, :])
              + b_ref[0])
    m = jnp.max(logits, axis=1, keepdims=True)
    lse = jnp.log(jnp.sum(jnp.exp(logits - m), axis=1, keepdims=True)) + m
    out_ref[...] = logits - lse


def _tc_postepi(agg, g, degp, bc_i, W_post, b_post):
    return pl.pallas_call(
        _postepi_body,
        grid=(8,),
        in_specs=[
            pl.BlockSpec((2, 1280, 128), lambda i: (0, i, 0)),
            pl.BlockSpec((2, 1280, 128), lambda i: (0, i, 0)),
            _DEGP_SPEC,
            pl.BlockSpec((1, 256), lambda i: (0, 0)),
            pl.BlockSpec((256, 64), lambda i: (0, 0)),
            pl.BlockSpec((1, 64), lambda i: (0, 0)),
        ],
        out_specs=[
            pl.BlockSpec((1280, 256), lambda i: (i, 0)),
            pl.BlockSpec((1280, 64), lambda i: (i, 0)),
        ],
        out_shape=[
            jax.ShapeDtypeStruct((NP, 256), F32),
            jax.ShapeDtypeStruct((NP, 64), F32),
        ],
    )(agg, g, degp, bc_i, W_post, b_post)


def kernel(x, edge_index, W_pre, b_pre, Wc, bc, W_post, b_post):
    xp = jnp.pad(x, ((0, NP - N), (0, 0)))
    ei = jnp.pad(edge_index, ((0, 0), (0, EP - E)), constant_values=N)
    src = ei[0].reshape(32, NB32, B)
    srcs2 = jnp.stack([src, src + NP])
    dsts = ei[1].reshape(32, NB32, B)

    degp = _sc_deg(dsts).reshape(2, NP, B)
    h0 = _tc_pre(xp, W_pre, b_pre.reshape(1, 256))

    g = _tc_mm1(h0, degp, Wc[0])
    for i in range(3):
        agg = _sc_scatter(g.reshape(2 * NP, B), srcs2, dsts).reshape(2, NP, B)
        if i < 2:
            g = _tc_fuse(agg, g, degp, bc[i].reshape(1, 256), Wc[i + 1])
    h, out = _tc_postepi(agg, g, degp, bc[2].reshape(1, 256), W_post,
                         b_post.reshape(1, 64))
    return (out[:N], h[:N], h[:N])


# speedup vs baseline: 1.0447x; 1.0447x over previous
import functools

import jax
import jax.numpy as jnp
from jax import lax
from jax.experimental import pallas as pl
from jax.experimental.pallas import tpu as pltpu
from jax.experimental.pallas import tpu_sc as plsc

N = 10000
NP = 10240
E = 320000
EP = 327680
NB32 = 80
CH = 16
B = 128
RT = NP // 16
F32 = jnp.float32

@functools.lru_cache(maxsize=1)
def _mesh():
    return plsc.VectorSubcoreMesh(core_axis_name="c", subcore_axis_name="s",
                                  num_cores=2, num_subcores=16)


def _zero_vmem(ref, nrows, ncols):
    z = jnp.zeros((16,), F32)
    def body(r, _):
        for j in range(ncols // 16):
            ref[r, pl.ds(j * 16, 16)] = z
        return 0
    lax.fori_loop(0, nrows, body, 0, unroll=False)


def _deg_body(dsts_hbm, out_hbm, dst_v, vals_v, dacc, sem):
    cid = lax.axis_index("c")
    sid = lax.axis_index("s")
    _zero_vmem(vals_v, B, B)
    for k in range(RT // B):
        pltpu.sync_copy(vals_v, dacc.at[pl.ds(sid * RT + k * B, B)])
    one = jnp.ones((16,), F32)
    def fill(r, _):
        for j in range(B // 16):
            vals_v[r, pl.ds(j * 16, 16)] = one
        return 0
    lax.fori_loop(0, B, fill, 0, unroll=False)
    pltpu.sync_copy(dsts_hbm.at[sid * 2 + cid], dst_v)
    plsc.subcore_barrier()
    def body(b, _):
        pltpu.sync_copy(vals_v, dacc.at[dst_v.at[b]], add=True)
        return 0
    lax.fori_loop(0, NB32, body, 0, unroll=False)
    plsc.subcore_barrier()
    pltpu.sync_copy(dacc.at[pl.ds(sid * RT, RT)],
                    out_hbm.at[pl.ds(cid * NP + sid * RT, RT)])


def _sc_deg(dsts):
    f = pl.kernel(
        _deg_body,
        out_type=jax.ShapeDtypeStruct((2 * NP, B), F32),
        mesh=_mesh(),
        scratch_types=[
            pltpu.VMEM((NB32, B), jnp.int32),
            pltpu.VMEM((B, B), F32),
            pltpu.VMEM_SHARED((NP, B), F32),
            pltpu.SemaphoreType.DMA,
        ],
    )
    return f(dsts)


def _scat_body(g_hbm, srcs_hbm, dsts_hbm, out_hbm, src_v, dst_v, rows_v, acc,
               sem_g0, sem_g1, sem_s0, sem_s1, sem_i):
    cid = lax.axis_index("c")
    sid = lax.axis_index("s")
    z = jnp.zeros((16,), F32)
    def zr(r, _):
        for j in range(B // 16):
            rows_v[0, r, pl.ds(j * 16, 16)] = z
        return 0
    lax.fori_loop(0, B, zr, 0, unroll=False)
    for k in range(RT // B):
        pltpu.sync_copy(rows_v.at[0], acc.at[pl.ds(sid * RT + k * B, B)])
    plsc.subcore_barrier()
    buf = (rows_v.at[0], rows_v.at[1])
    sg = (sem_g0, sem_g1)
    ss = (sem_s0, sem_s1)
    NCH = NB32 // CH
    TOT = 2 * NCH

    def load_idx(cc, slot):
        part = cc // NCH
        ch = cc % NCH
        pltpu.async_copy(
            srcs_hbm.at[cid, sid * 2 + part, pl.ds(ch * CH, CH)],
            src_v.at[slot], sem_i)
        pltpu.async_copy(
            dsts_hbm.at[sid * 2 + part, pl.ds(ch * CH, CH)],
            dst_v.at[slot], sem_i)

    def wait_idx(slot):
        pltpu.make_async_copy(dsts_hbm.at[0, pl.ds(0, CH)],
                              src_v.at[slot], sem_i).wait()
        pltpu.make_async_copy(dsts_hbm.at[0, pl.ds(0, CH)],
                              dst_v.at[slot], sem_i).wait()

    load_idx(0, 0)
    wait_idx(0)

    def chunk(cc, _):
        slot = cc % 2
        sv = src_v.at[slot]
        dv = dst_v.at[slot]
        @pl.when(cc + 1 < TOT)
        def _():
            load_idx(cc + 1, (cc + 1) % 2)
        pltpu.async_copy(g_hbm.at[sv.at[0]], buf[0], sg[0])
        def body(k2, _):
            k = 2 * k2
            @pl.when(k2 >= 1)
            def _():
                pltpu.make_async_copy(buf[1], acc.at[dv.at[k - 1]],
                                      ss[1]).wait()
            pltpu.async_copy(g_hbm.at[sv.at[k + 1]], buf[1], sg[1])
            pltpu.make_async_copy(g_hbm.at[sv.at[k]], buf[0], sg[0]).wait()
            pltpu.async_copy(buf[0], acc.at[dv.at[k]], ss[0], add=True)
            @pl.when(k2 < CH // 2 - 1)
            def _():
                pltpu.make_async_copy(buf[0], acc.at[dv.at[k]], ss[0]).wait()
                pltpu.async_copy(g_hbm.at[sv.at[k + 2]], buf[0], sg[0])
            pltpu.make_async_copy(g_hbm.at[sv.at[k + 1]], buf[1],
                                  sg[1]).wait()
            pltpu.async_copy(buf[1], acc.at[dv.at[k + 1]], ss[1], add=True)
            return 0
        lax.fori_loop(0, CH // 2, body, 0, unroll=False)
        pltpu.make_async_copy(buf[0], acc.at[dv.at[CH - 2]], ss[0]).wait()
        pltpu.make_async_copy(buf[1], acc.at[dv.at[CH - 1]], ss[1]).wait()
        @pl.when(cc + 1 < TOT)
        def _():
            wait_idx((cc + 1) % 2)
        return 0
    lax.fori_loop(0, TOT, chunk, 0, unroll=False)
    plsc.subcore_barrier()
    pltpu.sync_copy(acc.at[pl.ds(sid * RT, RT)],
                    out_hbm.at[pl.ds(cid * NP + sid * RT, RT)])


def _sc_scatter(g2, srcs2, dsts):
    f = pl.kernel(
        _scat_body,
        out_type=jax.ShapeDtypeStruct((2 * NP, B), F32),
        mesh=_mesh(),
        scratch_types=[
            pltpu.VMEM((2, CH, B), jnp.int32),
            pltpu.VMEM((2, CH, B), jnp.int32),
            pltpu.VMEM((2, B, B), F32),
            pltpu.VMEM_SHARED((NP, B), F32),
        ] + [pltpu.SemaphoreType.DMA] * 5,
    )
    return f(g2, srcs2, dsts)


def _prep_body(degp_ref, disb_ref):
    i = pl.program_id(0)
    d = degp_ref[0, :, 0:1] + degp_ref[1, :, 0:1]
    rows = lax.broadcasted_iota(jnp.int32, (1280, 1), 0) + i * 1280
    dis = jnp.where(rows < N, lax.rsqrt(d + 1.0), 0.0)
    disb_ref[...] = jnp.broadcast_to(dis, (1280, 128))


def _tc_prep(degp):
    return pl.pallas_call(
        _prep_body,
        grid=(8,),
        in_specs=[pl.BlockSpec((2, 1280, 128), lambda i: (0, i, 0))],
        out_specs=pl.BlockSpec((1280, 128), lambda i: (i, 0)),
        out_shape=jax.ShapeDtypeStruct((NP, 128), F32),
    )(degp)


_DOT = functools.partial(jnp.dot, preferred_element_type=F32,
                         precision=lax.Precision.HIGHEST)


def _pre_body(x_ref, w_ref, b_ref, h_ref):
    h = _DOT(x_ref[...], w_ref[...]) + b_ref[0]
    h_ref[0] = h[:, :128]
    h_ref[1] = h[:, 128:]


def _tc_pre(xp, W_pre, b_pre):
    return pl.pallas_call(
        _pre_body,
        grid=(8,),
        in_specs=[
            pl.BlockSpec((1280, 128), lambda i: (i, 0)),
            pl.BlockSpec((128, 256), lambda i: (0, 0)),
            pl.BlockSpec((1, 256), lambda i: (0, 0)),
        ],
        out_specs=pl.BlockSpec((2, 1280, 128), lambda i: (0, i, 0)),
        out_shape=jax.ShapeDtypeStruct((2, NP, 128), F32),
    )(xp, W_pre, b_pre)


def _mm1_body(h_ref, disb_ref, w_ref, g_ref):
    d = disb_ref[...]
    g = _DOT(h_ref[0] * d, w_ref[:128, :]) + _DOT(h_ref[1] * d, w_ref[128:, :])
    g_ref[0] = g[:, :128]
    g_ref[1] = g[:, 128:]


def _tc_mm1(h, disb, W):
    return pl.pallas_call(
        _mm1_body,
        grid=(8,),
        in_specs=[
            pl.BlockSpec((2, 1280, 128), lambda i: (0, i, 0)),
            pl.BlockSpec((1280, 128), lambda i: (i, 0)),
            pl.BlockSpec((256, 256), lambda i: (0, 0)),
        ],
        out_specs=pl.BlockSpec((2, 1280, 128), lambda i: (0, i, 0)),
        out_shape=jax.ShapeDtypeStruct((2, NP, 128), F32),
    )(h, disb, W)


def _fuse_body(agg_ref, g_ref, disb_ref, bc_ref, w_ref, gn_ref):
    d = disb_ref[...]
    t = []
    for j in range(2):
        u = jnp.maximum(d * (agg_ref[j] + g_ref[j])
                        + bc_ref[0, j * 128:(j + 1) * 128], 0.0)
        t.append(d * u)
    gn = _DOT(t[0], w_ref[:128, :]) + _DOT(t[1], w_ref[128:, :])
    gn_ref[0] = gn[:, :128]
    gn_ref[1] = gn[:, 128:]


def _tc_fuse(agg, g, disb, bc_i, W):
    return pl.pallas_call(
        _fuse_body,
        grid=(8,),
        in_specs=[
            pl.BlockSpec((2, 1280, 128), lambda i: (0, i, 0)),
            pl.BlockSpec((2, 1280, 128), lambda i: (0, i, 0)),
            pl.BlockSpec((1280, 128), lambda i: (i, 0)),
            pl.BlockSpec((1, 256), lambda i: (0, 0)),
            pl.BlockSpec((256, 256), lambda i: (0, 0)),
        ],
        out_specs=pl.BlockSpec((2, 1280, 128), lambda i: (0, i, 0)),
        out_shape=jax.ShapeDtypeStruct((2, NP, 128), F32),
    )(agg, g, disb, bc_i, W)


def _postepi_body(agg_ref, g_ref, disb_ref, bc_ref, w_ref, b_ref, h_ref,
                  out_ref):
    d = disb_ref[...]
    u = []
    for j in range(2):
        uj = jnp.maximum(d * (agg_ref[j] + g_ref[j])
                         + bc_ref[0, j * 128:(j + 1) * 128], 0.0)
        h_ref[:, j * 128:(j + 1) * 128] = uj
        u.append(uj)
    logits = (_DOT(u[0], w_ref[:128, :]) + _DOT(u[1], w_ref[128:, :])
              + b_ref[0])
    m = jnp.max(logits, axis=1, keepdims=True)
    lse = jnp.log(jnp.sum(jnp.exp(logits - m), axis=1, keepdims=True)) + m
    out_ref[...] = logits - lse


def _tc_postepi(agg, g, disb, bc_i, W_post, b_post):
    return pl.pallas_call(
        _postepi_body,
        grid=(8,),
        in_specs=[
            pl.BlockSpec((2, 1280, 128), lambda i: (0, i, 0)),
            pl.BlockSpec((2, 1280, 128), lambda i: (0, i, 0)),
            pl.BlockSpec((1280, 128), lambda i: (i, 0)),
            pl.BlockSpec((1, 256), lambda i: (0, 0)),
            pl.BlockSpec((256, 64), lambda i: (0, 0)),
            pl.BlockSpec((1, 64), lambda i: (0, 0)),
        ],
        out_specs=[
            pl.BlockSpec((1280, 256), lambda i: (i, 0)),
            pl.BlockSpec((1280, 64), lambda i: (i, 0)),
        ],
        out_shape=[
            jax.ShapeDtypeStruct((NP, 256), F32),
            jax.ShapeDtypeStruct((NP, 64), F32),
        ],
    )(agg, g, disb, bc_i, W_post, b_post)


def kernel(x, edge_index, W_pre, b_pre, Wc, bc, W_post, b_post):
    xp = jnp.pad(x, ((0, NP - N), (0, 0)))
    ei = jnp.pad(edge_index, ((0, 0), (0, EP - E)), constant_values=N)
    src = ei[0].reshape(32, NB32, B)
    srcs2 = jnp.stack([src, src + NP])
    dsts = ei[1].reshape(32, NB32, B)

    degp = _sc_deg(dsts).reshape(2, NP, B)
    disb = _tc_prep(degp)
    h0 = _tc_pre(xp, W_pre, b_pre.reshape(1, 256))

    g = _tc_mm1(h0, disb, Wc[0])
    for i in range(3):
        agg = _sc_scatter(g.reshape(2 * NP, B), srcs2, dsts).reshape(2, NP, B)
        if i < 2:
            g = _tc_fuse(agg, g, disb, bc[i].reshape(1, 256), Wc[i + 1])
    h, out = _tc_postepi(agg, g, disb, bc[2].reshape(1, 256), W_post,
                         b_post.reshape(1, 64))
    return (out[:N], h[:N], h[:N])


# speedup vs baseline: 1.0461x; 1.0013x over previous
import functools

import jax
import jax.numpy as jnp
from jax import lax
from jax.experimental import pallas as pl
from jax.experimental.pallas import tpu as pltpu
from jax.experimental.pallas import tpu_sc as plsc

N = 10000
NP = 10240
E = 320000
EP = 327680
NB32 = 80
CH = 16
B = 128
RT = NP // 16
F32 = jnp.float32

@functools.lru_cache(maxsize=1)
def _mesh():
    return plsc.VectorSubcoreMesh(core_axis_name="c", subcore_axis_name="s",
                                  num_cores=2, num_subcores=16)


def _zero_vmem(ref, nrows, ncols):
    z = jnp.zeros((16,), F32)
    def body(r, _):
        for j in range(ncols // 16):
            ref[r, pl.ds(j * 16, 16)] = z
        return 0
    lax.fori_loop(0, nrows, body, 0, unroll=False)


def _deg_body(dsts_hbm, out_hbm, dst_v, vals_v, dacc, sem):
    cid = lax.axis_index("c")
    sid = lax.axis_index("s")
    _zero_vmem(vals_v, B, B)
    for k in range(RT // B):
        pltpu.sync_copy(vals_v, dacc.at[pl.ds(sid * RT + k * B, B)])
    one = jnp.ones((16,), F32)
    def fill(r, _):
        for j in range(B // 16):
            vals_v[r, pl.ds(j * 16, 16)] = one
        return 0
    lax.fori_loop(0, B, fill, 0, unroll=False)
    pltpu.sync_copy(dsts_hbm.at[sid * 2 + cid], dst_v)
    plsc.subcore_barrier()
    GD = 8
    def body(grp, _):
        for j in range(GD):
            pltpu.async_copy(vals_v, dacc.at[dst_v.at[grp * GD + j]], sem,
                             add=True)
        @pl.when(grp >= 1)
        def _():
            for j in range(GD):
                pltpu.make_async_copy(
                    vals_v, dacc.at[dst_v.at[(grp - 1) * GD + j]],
                    sem).wait()
        return 0
    lax.fori_loop(0, NB32 // GD, body, 0, unroll=False)
    for j in range(GD):
        pltpu.make_async_copy(vals_v, dacc.at[dst_v.at[NB32 - GD + j]],
                              sem).wait()
    plsc.subcore_barrier()
    pltpu.sync_copy(dacc.at[pl.ds(sid * RT, RT)],
                    out_hbm.at[pl.ds(cid * NP + sid * RT, RT)])


def _sc_deg(dsts):
    f = pl.kernel(
        _deg_body,
        out_type=jax.ShapeDtypeStruct((2 * NP, B), F32),
        mesh=_mesh(),
        scratch_types=[
            pltpu.VMEM((NB32, B), jnp.int32),
            pltpu.VMEM((B, B), F32),
            pltpu.VMEM_SHARED((NP, B), F32),
            pltpu.SemaphoreType.DMA,
        ],
    )
    return f(dsts)


def _scat_body(g_hbm, srcs_hbm, dsts_hbm, out_hbm, src_v, dst_v, rows_v, acc,
               sem_g0, sem_g1, sem_s0, sem_s1, sem_i):
    cid = lax.axis_index("c")
    sid = lax.axis_index("s")
    z = jnp.zeros((16,), F32)
    def zr(r, _):
        for j in range(B // 16):
            rows_v[0, r, pl.ds(j * 16, 16)] = z
        return 0
    lax.fori_loop(0, B, zr, 0, unroll=False)
    for k in range(RT // B):
        pltpu.sync_copy(rows_v.at[0], acc.at[pl.ds(sid * RT + k * B, B)])
    plsc.subcore_barrier()
    buf = (rows_v.at[0], rows_v.at[1])
    sg = (sem_g0, sem_g1)
    ss = (sem_s0, sem_s1)
    NCH = NB32 // CH
    TOT = 2 * NCH

    def load_idx(cc, slot):
        part = cc // NCH
        ch = cc % NCH
        pltpu.async_copy(
            srcs_hbm.at[cid, sid * 2 + part, pl.ds(ch * CH, CH)],
            src_v.at[slot], sem_i)
        pltpu.async_copy(
            dsts_hbm.at[sid * 2 + part, pl.ds(ch * CH, CH)],
            dst_v.at[slot], sem_i)

    def wait_idx(slot):
        pltpu.make_async_copy(dsts_hbm.at[0, pl.ds(0, CH)],
                              src_v.at[slot], sem_i).wait()
        pltpu.make_async_copy(dsts_hbm.at[0, pl.ds(0, CH)],
                              dst_v.at[slot], sem_i).wait()

    load_idx(0, 0)
    wait_idx(0)

    def chunk(cc, _):
        slot = cc % 2
        sv = src_v.at[slot]
        dv = dst_v.at[slot]
        @pl.when(cc + 1 < TOT)
        def _():
            load_idx(cc + 1, (cc + 1) % 2)
        pltpu.async_copy(g_hbm.at[sv.at[0]], buf[0], sg[0])
        def body(k2, _):
            k = 2 * k2
            @pl.when(k2 >= 1)
            def _():
                pltpu.make_async_copy(buf[1], acc.at[dv.at[k - 1]],
                                      ss[1]).wait()
            pltpu.async_copy(g_hbm.at[sv.at[k + 1]], buf[1], sg[1])
            pltpu.make_async_copy(g_hbm.at[sv.at[k]], buf[0], sg[0]).wait()
            pltpu.async_copy(buf[0], acc.at[dv.at[k]], ss[0], add=True)
            @pl.when(k2 < CH // 2 - 1)
            def _():
                pltpu.make_async_copy(buf[0], acc.at[dv.at[k]], ss[0]).wait()
                pltpu.async_copy(g_hbm.at[sv.at[k + 2]], buf[0], sg[0])
            pltpu.make_async_copy(g_hbm.at[sv.at[k + 1]], buf[1],
                                  sg[1]).wait()
            pltpu.async_copy(buf[1], acc.at[dv.at[k + 1]], ss[1], add=True)
            return 0
        lax.fori_loop(0, CH // 2, body, 0, unroll=False)
        pltpu.make_async_copy(buf[0], acc.at[dv.at[CH - 2]], ss[0]).wait()
        pltpu.make_async_copy(buf[1], acc.at[dv.at[CH - 1]], ss[1]).wait()
        @pl.when(cc + 1 < TOT)
        def _():
            wait_idx((cc + 1) % 2)
        return 0
    lax.fori_loop(0, TOT, chunk, 0, unroll=False)
    plsc.subcore_barrier()
    pltpu.sync_copy(acc.at[pl.ds(sid * RT, RT)],
                    out_hbm.at[pl.ds(cid * NP + sid * RT, RT)])


def _sc_scatter(g2, srcs2, dsts):
    f = pl.kernel(
        _scat_body,
        out_type=jax.ShapeDtypeStruct((2 * NP, B), F32),
        mesh=_mesh(),
        scratch_types=[
            pltpu.VMEM((2, CH, B), jnp.int32),
            pltpu.VMEM((2, CH, B), jnp.int32),
            pltpu.VMEM((2, B, B), F32),
            pltpu.VMEM_SHARED((NP, B), F32),
        ] + [pltpu.SemaphoreType.DMA] * 5,
    )
    return f(g2, srcs2, dsts)


def _prep_body(degp_ref, disb_ref):
    i = pl.program_id(0)
    d = degp_ref[0, :, 0:1] + degp_ref[1, :, 0:1]
    rows = lax.broadcasted_iota(jnp.int32, (1280, 1), 0) + i * 1280
    dis = jnp.where(rows < N, lax.rsqrt(d + 1.0), 0.0)
    disb_ref[...] = jnp.broadcast_to(dis, (1280, 128))


def _tc_prep(degp):
    return pl.pallas_call(
        _prep_body,
        grid=(8,),
        in_specs=[pl.BlockSpec((2, 1280, 128), lambda i: (0, i, 0))],
        out_specs=pl.BlockSpec((1280, 128), lambda i: (i, 0)),
        out_shape=jax.ShapeDtypeStruct((NP, 128), F32),
    )(degp)


_DOT = functools.partial(jnp.dot, preferred_element_type=F32,
                         precision=lax.Precision.HIGHEST)


def _pre_body(x_ref, w_ref, b_ref, h_ref):
    h = _DOT(x_ref[...], w_ref[...]) + b_ref[0]
    h_ref[0] = h[:, :128]
    h_ref[1] = h[:, 128:]


def _tc_pre(xp, W_pre, b_pre):
    return pl.pallas_call(
        _pre_body,
        grid=(8,),
        in_specs=[
            pl.BlockSpec((1280, 128), lambda i: (i, 0)),
            pl.BlockSpec((128, 256), lambda i: (0, 0)),
            pl.BlockSpec((1, 256), lambda i: (0, 0)),
        ],
        out_specs=pl.BlockSpec((2, 1280, 128), lambda i: (0, i, 0)),
        out_shape=jax.ShapeDtypeStruct((2, NP, 128), F32),
    )(xp, W_pre, b_pre)


def _mm1_body(h_ref, disb_ref, w_ref, g_ref):
    d = disb_ref[...]
    g = _DOT(h_ref[0] * d, w_ref[:128, :]) + _DOT(h_ref[1] * d, w_ref[128:, :])
    g_ref[0] = g[:, :128]
    g_ref[1] = g[:, 128:]


def _tc_mm1(h, disb, W):
    return pl.pallas_call(
        _mm1_body,
        grid=(8,),
        in_specs=[
            pl.BlockSpec((2, 1280, 128), lambda i: (0, i, 0)),
            pl.BlockSpec((1280, 128), lambda i: (i, 0)),
            pl.BlockSpec((256, 256), lambda i: (0, 0)),
        ],
        out_specs=pl.BlockSpec((2, 1280, 128), lambda i: (0, i, 0)),
        out_shape=jax.ShapeDtypeStruct((2, NP, 128), F32),
    )(h, disb, W)


def _fuse_body(agg_ref, g_ref, disb_ref, bc_ref, w_ref, gn_ref):
    d = disb_ref[...]
    t = []
    for j in range(2):
        u = jnp.maximum(d * (agg_ref[j] + g_ref[j])
                        + bc_ref[0, j * 128:(j + 1) * 128], 0.0)
        t.append(d * u)
    gn = _DOT(t[0], w_ref[:128, :]) + _DOT(t[1], w_ref[128:, :])
    gn_ref[0] = gn[:, :128]
    gn_ref[1] = gn[:, 128:]


def _tc_fuse(agg, g, disb, bc_i, W):
    return pl.pallas_call(
        _fuse_body,
        grid=(8,),
        in_specs=[
            pl.BlockSpec((2, 1280, 128), lambda i: (0, i, 0)),
            pl.BlockSpec((2, 1280, 128), lambda i: (0, i, 0)),
            pl.BlockSpec((1280, 128), lambda i: (i, 0)),
            pl.BlockSpec((1, 256), lambda i: (0, 0)),
            pl.BlockSpec((256, 256), lambda i: (0, 0)),
        ],
        out_specs=pl.BlockSpec((2, 1280, 128), lambda i: (0, i, 0)),
        out_shape=jax.ShapeDtypeStruct((2, NP, 128), F32),
    )(agg, g, disb, bc_i, W)


def _postepi_body(agg_ref, g_ref, disb_ref, bc_ref, w_ref, b_ref, h_ref,
                  out_ref):
    d = disb_ref[...]
    u = []
    for j in range(2):
        uj = jnp.maximum(d * (agg_ref[j] + g_ref[j])
                         + bc_ref[0, j * 128:(j + 1) * 128], 0.0)
        h_ref[:, j * 128:(j + 1) * 128] = uj
        u.append(uj)
    logits = (_DOT(u[0], w_ref[:128, :]) + _DOT(u[1], w_ref[128:, :])
              + b_ref[0])
    m = jnp.max(logits, axis=1, keepdims=True)
    lse = jnp.log(jnp.sum(jnp.exp(logits - m), axis=1, keepdims=True)) + m
    out_ref[...] = logits - lse


def _tc_postepi(agg, g, disb, bc_i, W_post, b_post):
    return pl.pallas_call(
        _postepi_body,
        grid=(8,),
        in_specs=[
            pl.BlockSpec((2, 1280, 128), lambda i: (0, i, 0)),
            pl.BlockSpec((2, 1280, 128), lambda i: (0, i, 0)),
            pl.BlockSpec((1280, 128), lambda i: (i, 0)),
            pl.BlockSpec((1, 256), lambda i: (0, 0)),
            pl.BlockSpec((256, 64), lambda i: (0, 0)),
            pl.BlockSpec((1, 64), lambda i: (0, 0)),
        ],
        out_specs=[
            pl.BlockSpec((1280, 256), lambda i: (i, 0)),
            pl.BlockSpec((1280, 64), lambda i: (i, 0)),
        ],
        out_shape=[
            jax.ShapeDtypeStruct((NP, 256), F32),
            jax.ShapeDtypeStruct((NP, 64), F32),
        ],
    )(agg, g, disb, bc_i, W_post, b_post)


def kernel(x, edge_index, W_pre, b_pre, Wc, bc, W_post, b_post):
    xp = jnp.pad(x, ((0, NP - N), (0, 0)))
    ei = jnp.pad(edge_index, ((0, 0), (0, EP - E)), constant_values=N)
    src = ei[0].reshape(32, NB32, B)
    srcs2 = jnp.stack([src, src + NP])
    dsts = ei[1].reshape(32, NB32, B)

    degp = _sc_deg(dsts).reshape(2, NP, B)
    disb = _tc_prep(degp)
    h0 = _tc_pre(xp, W_pre, b_pre.reshape(1, 256))

    g = _tc_mm1(h0, disb, Wc[0])
    for i in range(3):
        agg = _sc_scatter(g.reshape(2 * NP, B), srcs2, dsts).reshape(2, NP, B)
        if i < 2:
            g = _tc_fuse(agg, g, disb, bc[i].reshape(1, 256), Wc[i + 1])
    h, out = _tc_postepi(agg, g, disb, bc[2].reshape(1, 256), W_post,
                         b_post.reshape(1, 64))
    return (out[:N], h[:N], h[:N])
